# Initial kernel scaffold; baseline (speedup 1.0000x reference)
import jax
import jax.numpy as jnp
from jax.experimental import pallas as pl


def kernel(x, frame_rate, proj_in_w, proj_in_b, proj_out_w, proj_out_b, codebooks):
    raise NotImplementedError("write your pallas kernel here")


# speedup vs baseline: 3.2001x; 3.2001x over previous
import math

import jax
import jax.numpy as jnp
from jax.experimental import pallas as pl

N_Q = 8
BINS = 1024
DIM = 256
CODE_DIM = 8


def _rvq_kernel(x_ref, pw_ref, pb_ref, pow_ref, pob_ref, cb_ref, c2_ref,
                q_out_ref, codes_ref, loss_ref):
    b = pl.program_id(0)
    t = pl.program_id(1)
    Tt = x_ref.shape[2]

    r = x_ref[0]
    qacc = jnp.zeros_like(r)
    row_iota = jax.lax.broadcasted_iota(jnp.int32, (BINS, Tt), 0)

    idx_list = []
    loss_parts = []
    for i in range(N_Q):
        P = pw_ref[i]
        xp = jax.lax.dot_general(P, r, (((1,), (0,)), ((), ())),
                                 preferred_element_type=jnp.float32)
        xp = xp + pb_ref[i][:, None]
        sc = jax.lax.dot_general(cb_ref[i], xp, (((1,), (0,)), ((), ())),
                                 preferred_element_type=jnp.float32)
        sc = c2_ref[i][:, None] - 2.0 * sc
        idx = jnp.argmin(sc, axis=0)
        onehot = (row_iota == idx[None, :]).astype(jnp.float32)
        q = jax.lax.dot_general(cb_ref[i], onehot, (((0,), (0,)), ((), ())),
                                preferred_element_type=jnp.float32)
        loss_parts.append(jnp.sum((q - xp) ** 2, axis=0))
        qo = jax.lax.dot_general(pow_ref[i], q, (((1,), (0,)), ((), ())),
                                 preferred_element_type=jnp.float32)
        qo = qo + pob_ref[i][:, None]
        r = r - qo
        qacc = qacc + qo
        idx_list.append(idx)

    q_out_ref[0] = qacc
    codes_ref[0] = jnp.stack(idx_list, axis=0)
    loss_ref[0] = jnp.stack(loss_parts, axis=0)


def kernel(x, frame_rate, proj_in_w, proj_in_b, proj_out_w, proj_out_b, codebooks):
    B, D, T = x.shape
    Tt = 512
    grid = (B, T // Tt)

    c2 = jnp.sum(codebooks * codebooks, axis=-1)

    quantized, codes_tmp, loss_acc = pl.pallas_call(
        _rvq_kernel,
        grid=grid,
        in_specs=[
            pl.BlockSpec((1, D, Tt), lambda b, t: (b, 0, t)),
            pl.BlockSpec((N_Q, CODE_DIM, D), lambda b, t: (0, 0, 0)),
            pl.BlockSpec((N_Q, CODE_DIM), lambda b, t: (0, 0)),
            pl.BlockSpec((N_Q, D, CODE_DIM), lambda b, t: (0, 0, 0)),
            pl.BlockSpec((N_Q, D), lambda b, t: (0, 0)),
            pl.BlockSpec((N_Q, BINS, CODE_DIM), lambda b, t: (0, 0, 0)),
            pl.BlockSpec((N_Q, BINS), lambda b, t: (0, 0)),
        ],
        out_specs=[
            pl.BlockSpec((1, D, Tt), lambda b, t: (b, 0, t)),
            pl.BlockSpec((1, N_Q, Tt), lambda b, t: (b, 0, t)),
            pl.BlockSpec((1, N_Q, Tt), lambda b, t: (b, 0, t)),
        ],
        out_shape=[
            jax.ShapeDtypeStruct((B, D, T), jnp.float32),
            jax.ShapeDtypeStruct((B, N_Q, T), jnp.int32),
            jax.ShapeDtypeStruct((B, N_Q, T), jnp.float32),
        ],
    )(x, proj_in_w, proj_in_b, proj_out_w, proj_out_b, codebooks, c2)

    codes = jnp.transpose(codes_tmp, (1, 0, 2))
    commit_loss = jnp.sum(loss_acc, axis=(0, 2)) / (B * T * CODE_DIM)
    bw = jnp.asarray(N_Q * math.log2(BINS) * frame_rate, x.dtype)
    return quantized, codes, bw, commit_loss


# speedup vs baseline: 3.8235x; 1.1948x over previous
import math

import jax
import jax.numpy as jnp
from jax.experimental import pallas as pl

N_Q = 8
BINS = 1024
DIM = 256
CODE_DIM = 8


def _rvq_kernel(x_ref, pw_ref, pb_ref, pow_ref, pob_ref, cba_ref,
                q_out_ref, codes_ref, loss_ref):
    Tt = x_ref.shape[2]
    H = Tt // 2
    row_iota = jax.lax.broadcasted_iota(jnp.int32, (BINS, H), 0)

    def stage(i, r, ones):
        P = pw_ref[i]
        xp = jax.lax.dot_general(P, r, (((1,), (0,)), ((), ())),
                                 preferred_element_type=jnp.float32)
        xp = xp + pb_ref[i][:, None]
        xpaug = jnp.concatenate([xp, ones], axis=0)
        sc = jax.lax.dot_general(cba_ref[i], xpaug, (((1,), (0,)), ((), ())),
                                 preferred_element_type=jnp.float32)
        idx = jnp.argmin(sc, axis=0)
        onehot = (row_iota == idx[None, :]).astype(jnp.float32)
        qm2 = jax.lax.dot_general(cba_ref[i, :, :CODE_DIM], onehot,
                                  (((0,), (0,)), ((), ())),
                                  preferred_element_type=jnp.float32)
        q = -0.5 * qm2
        lp = jnp.sum((q - xp) ** 2, axis=0)
        qo = jax.lax.dot_general(pow_ref[i], q, (((1,), (0,)), ((), ())),
                                 preferred_element_type=jnp.float32)
        qo = qo + pob_ref[i][:, None]
        return r - qo, qo, idx, lp

    halves = []
    for h in range(2):
        r = x_ref[0, :, h * H:(h + 1) * H]
        halves.append({"r": r, "qacc": jnp.zeros_like(r), "idx": [], "lp": []})
    ones = jnp.ones((1, H), jnp.float32)

    for i in range(N_Q):
        for st in halves:
            r, qo, idx, lp = stage(i, st["r"], ones)
            st["r"] = r
            st["qacc"] = st["qacc"] + qo
            st["idx"].append(idx)
            st["lp"].append(lp)

    for h, st in enumerate(halves):
        sl = pl.ds(h * H, H)
        q_out_ref[0, :, sl] = st["qacc"]
        codes_ref[0, :, sl] = jnp.stack(st["idx"], axis=0)
        loss_ref[0, :, sl] = jnp.stack(st["lp"], axis=0)


def kernel(x, frame_rate, proj_in_w, proj_in_b, proj_out_w, proj_out_b, codebooks):
    B, D, T = x.shape
    Tt = 1024
    grid = (B, T // Tt)

    c2 = jnp.sum(codebooks * codebooks, axis=-1)
    cb_aug = jnp.concatenate([-2.0 * codebooks, c2[:, :, None]], axis=-1)

    quantized, codes_tmp, loss_parts = pl.pallas_call(
        _rvq_kernel,
        grid=grid,
        in_specs=[
            pl.BlockSpec((1, D, Tt), lambda b, t: (b, 0, t)),
            pl.BlockSpec((N_Q, CODE_DIM, D), lambda b, t: (0, 0, 0)),
            pl.BlockSpec((N_Q, CODE_DIM), lambda b, t: (0, 0)),
            pl.BlockSpec((N_Q, D, CODE_DIM), lambda b, t: (0, 0, 0)),
            pl.BlockSpec((N_Q, D), lambda b, t: (0, 0)),
            pl.BlockSpec((N_Q, BINS, CODE_DIM + 1), lambda b, t: (0, 0, 0)),
        ],
        out_specs=[
            pl.BlockSpec((1, D, Tt), lambda b, t: (b, 0, t)),
            pl.BlockSpec((1, N_Q, Tt), lambda b, t: (b, 0, t)),
            pl.BlockSpec((1, N_Q, Tt), lambda b, t: (b, 0, t)),
        ],
        out_shape=[
            jax.ShapeDtypeStruct((B, D, T), jnp.float32),
            jax.ShapeDtypeStruct((B, N_Q, T), jnp.int32),
            jax.ShapeDtypeStruct((B, N_Q, T), jnp.float32),
        ],
    )(x, proj_in_w, proj_in_b, proj_out_w, proj_out_b, cb_aug)

    codes = jnp.transpose(codes_tmp, (1, 0, 2))
    commit_loss = jnp.sum(loss_parts, axis=(0, 2)) / (B * T * CODE_DIM)
    bw = jnp.asarray(N_Q * math.log2(BINS) * frame_rate, x.dtype)
    return quantized, codes, bw, commit_loss
